# baseline (speedup 1.0000x reference)
import functools

import jax
import jax.numpy as jnp
from jax import lax
from jax.experimental import pallas as pl
from jax.experimental.pallas import tpu as pltpu
from jax.experimental.pallas import tpu_sc as plsc

_CHUNK = 128
_K = 4
_NBUF = 2


def _emb_call(idx, table, n, d, nw, n_chunks, n_super, num_cores):
    per_w = n // nw
    mesh = plsc.VectorSubcoreMesh(core_axis_name="c", subcore_axis_name="s")

    @functools.partial(
        pl.kernel,
        mesh=mesh,
        compiler_params=pltpu.CompilerParams(use_tc_tiling_on_sc=False),
        out_type=jax.ShapeDtypeStruct((n, d), jnp.float32),
        scratch_types=[
            pltpu.VMEM((n_chunks, _CHUNK), jnp.int32),
            pltpu.VMEM((_NBUF, _K * _CHUNK, d), jnp.float32),
            pltpu.SemaphoreType.DMA,
        ],
    )
    def emb(idx_hbm, table_hbm, out_hbm, idx_v, rows_v, gsem):
        wid = lax.axis_index("s") * num_cores + lax.axis_index("c")
        base = wid * per_w
        pltpu.sync_copy(idx_hbm.at[wid], idx_v)

        def fire(sc, u):
            for j in range(_K):
                pltpu.async_copy(
                    table_hbm.at[idx_v.at[sc * _K + j]],
                    rows_v.at[u, pl.ds(j * _CHUNK, _CHUNK)],
                    gsem,
                )

        def drain(u):
            pltpu.make_async_copy(
                table_hbm.at[pl.ds(0, _K * _CHUNK)],
                rows_v.at[u],
                gsem,
            ).wait()

        def put(sc, u):
            pltpu.sync_copy(
                rows_v.at[u],
                out_hbm.at[pl.ds(base + sc * (_K * _CHUNK), _K * _CHUNK)],
            )

        for u in range(_NBUF):
            fire(u, u)

        def body(i, carry):
            c0 = i * _NBUF
            for u in range(_NBUF):
                drain(u)
                put(c0 + u, u)
                fire(c0 + u + _NBUF, u)
            return carry

        lax.fori_loop(0, (n_super - _NBUF) // _NBUF, body, 0)

        for u in range(_NBUF):
            drain(u)
            put(n_super - _NBUF + u, u)

    return emb(idx, table)


@jax.jit
def kernel(source_ids, table):
    b, s = source_ids.shape
    n = b * s
    d = table.shape[1]
    info = plsc.get_sparse_core_info()
    nw = info.num_cores * info.num_subcores
    per_w = n // nw
    n_chunks = per_w // _CHUNK
    n_super = n_chunks // _K
    idx = jnp.reshape(source_ids.astype(jnp.int32), (nw, n_chunks, _CHUNK))
    out = _emb_call(idx, table, n, d, nw, n_chunks, n_super, info.num_cores)
    return jnp.reshape(out, (b, s, d))


# speedup vs baseline: 1.0041x; 1.0041x over previous
import functools

import jax
import jax.numpy as jnp
from jax import lax
from jax.experimental import pallas as pl
from jax.experimental.pallas import tpu as pltpu
from jax.experimental.pallas import tpu_sc as plsc

_CHUNK = 128
_K = 2
_NBUF = 4
_LAG = 2


def _emb_call(idx, table, n, d, nw, n_chunks, n_super, num_cores):
    per_w = n // nw
    rows = _K * _CHUNK
    mesh = plsc.VectorSubcoreMesh(core_axis_name="c", subcore_axis_name="s")

    @functools.partial(
        pl.kernel,
        mesh=mesh,
        compiler_params=pltpu.CompilerParams(use_tc_tiling_on_sc=False),
        out_type=jax.ShapeDtypeStruct((n, d), jnp.float32),
        scratch_types=[
            pltpu.VMEM((n_chunks, _CHUNK), jnp.int32),
            pltpu.VMEM((_NBUF, rows, d), jnp.float32),
            pltpu.SemaphoreType.DMA,
            pltpu.SemaphoreType.DMA,
        ],
    )
    def emb(idx_hbm, table_hbm, out_hbm, idx_v, rows_v, gsem, osem):
        wid = lax.axis_index("s") * num_cores + lax.axis_index("c")
        base = wid * per_w
        pltpu.sync_copy(idx_hbm.at[wid], idx_v)

        def fire(sc, u):
            for j in range(_K):
                pltpu.async_copy(
                    table_hbm.at[idx_v.at[sc * _K + j]],
                    rows_v.at[u, pl.ds(j * _CHUNK, _CHUNK)],
                    gsem,
                )

        def drain_gather(u):
            pltpu.make_async_copy(
                table_hbm.at[pl.ds(0, rows)], rows_v.at[u], gsem
            ).wait()

        def put(sc, u):
            pltpu.async_copy(
                rows_v.at[u], out_hbm.at[pl.ds(base + sc * rows, rows)], osem
            )

        def wait_put_one():
            pltpu.make_async_copy(
                table_hbm.at[pl.ds(0, rows)], rows_v.at[0], osem
            ).wait()

        for c in range(_LAG):
            fire(c, c % _NBUF)
        for c in range(_LAG, _NBUF):
            fire(c, c % _NBUF)
            drain_gather((c - _LAG) % _NBUF)
            put(c - _LAG, (c - _LAG) % _NBUF)

        def body(i, carry):
            c0 = _NBUF + i * _NBUF
            for j in range(_NBUF):
                c = c0 + j
                wait_put_one()
                fire(c, j)
                drain_gather((j - _LAG) % _NBUF)
                put(c - _LAG, (j - _LAG) % _NBUF)
            return carry

        lax.fori_loop(0, (n_super - _NBUF) // _NBUF, body, 0)

        for c in range(n_super, n_super + _LAG):
            u = (c - _LAG) % _NBUF
            drain_gather(u)
            put(c - _LAG, u)
        for _ in range(_NBUF):
            wait_put_one()

    return emb(idx, table)


@jax.jit
def kernel(source_ids, table):
    b, s = source_ids.shape
    n = b * s
    d = table.shape[1]
    info = plsc.get_sparse_core_info()
    nw = info.num_cores * info.num_subcores
    per_w = n // nw
    n_chunks = per_w // _CHUNK
    n_super = n_chunks // _K
    idx = jnp.reshape(source_ids.astype(jnp.int32), (nw, n_chunks, _CHUNK))
    out = _emb_call(idx, table, n, d, nw, n_chunks, n_super, info.num_cores)
    return jnp.reshape(out, (b, s, d))
